# baseline (speedup 1.0000x reference)
import functools

import jax
import jax.numpy as jnp
from jax import lax
from jax.experimental import pallas as pl
from jax.experimental.pallas import tpu as pltpu
from jax.experimental.pallas import tpu_sc as plsc

N = 10000
E = 320000
D = 128
EPS = 1e-5

NC = 2
NS = 16
NW = NC * NS
CHUNK = 40
EPW = E // NW
NCHUNK = EPW // CHUNK
NPAD = 10112
RPS = NPAD // NS
NBUF = 3


def _seg_sum_body(x_hbm, src_hbm, dst_hbm, out_hbm, idx, rows, aggsh, *sems):
    c = lax.axis_index("c")
    s = lax.axis_index("s")
    wid = c * NS + s

    gsem = list(sems[:NBUF])
    dsem = list(sems[NBUF:2 * NBUF])
    ssem = list(sems[2 * NBUF:])
    rbuf = [rows.at[t] for t in range(NBUF)]
    dslot = [idx.at[NCHUNK + t] for t in range(NBUF)]

    zero = jnp.zeros((16,), jnp.float32)

    def zrow(i, carry):
        for j in range(D // 16):
            rows[0, i, pl.ds(j * 16, 16)] = zero
        return carry

    lax.fori_loop(0, CHUNK, zrow, 0)
    zbase = s * RPS

    def zcopy(t, carry):
        pltpu.sync_copy(rbuf[0], aggsh.at[pl.ds(zbase + t * CHUNK, CHUNK)])
        return carry

    lax.fori_loop(0, RPS // CHUNK, zcopy, 0)
    if RPS % CHUNK:
        pltpu.sync_copy(rbuf[0].at[pl.ds(0, RPS % CHUNK)],
                        aggsh.at[pl.ds(zbase + (RPS // CHUNK) * CHUNK,
                                       RPS % CHUNK)])

    pltpu.sync_copy(src_hbm.at[wid], idx.at[pl.ds(0, NCHUNK)])
    plsc.subcore_barrier()

    def dstart(j, t):
        pltpu.async_copy(dst_hbm.at[wid, j], dslot[t], dsem[t])

    def dwait(j, t):
        pltpu.make_async_copy(dst_hbm.at[wid, j], dslot[t], dsem[t]).wait()

    def gstart(j, t):
        pltpu.async_copy(x_hbm.at[idx.at[j]], rbuf[t], gsem[t])

    def gwait(j, t):
        pltpu.make_async_copy(x_hbm.at[idx.at[j]], rbuf[t], gsem[t]).wait()

    def sstart(j, t):
        pltpu.async_copy(rbuf[t], aggsh.at[dslot[t]], ssem[t], add=True)

    def swait(j, t):
        pltpu.make_async_copy(rbuf[t], aggsh.at[dslot[t]], ssem[t]).wait()

    def visit(j, t, first=False, last=False):
        tn = (t + 2) % NBUF
        gwait(j, t)
        dwait(j, t)
        sstart(j, t)
        if not first:
            swait(j - 1, tn)
        if not last:
            dstart(j + 2, tn)
            gstart(j + 2, tn)

    dstart(0, 0)
    gstart(0, 0)
    dstart(1, 1)
    gstart(1, 1)
    visit(0, 0, first=True)
    visit(1, 1)

    def body(k, carry):
        j0 = NBUF * k + 2
        for i in range(NBUF):
            visit(j0 + i, (2 + i) % NBUF)
        return carry

    nbody = (NCHUNK - 4) // NBUF
    lax.fori_loop(0, nbody, body, 0)
    visit(NCHUNK - 2, (NCHUNK - 2) % NBUF, last=True)
    visit(NCHUNK - 1, (NCHUNK - 1) % NBUF, last=True)
    swait(NCHUNK - 1, (NCHUNK - 1) % NBUF)
    plsc.subcore_barrier()

    pltpu.sync_copy(aggsh.at[pl.ds(s * RPS, RPS)],
                    out_hbm.at[c, pl.ds(s * RPS, RPS)])


@functools.partial(
    pl.kernel,
    out_type=jax.ShapeDtypeStruct((NC, NPAD, D), jnp.float32),
    mesh=plsc.VectorSubcoreMesh(core_axis_name="c", subcore_axis_name="s"),
    scratch_types=[
        pltpu.VMEM((NCHUNK + NBUF, CHUNK), jnp.int32),
        pltpu.VMEM((NBUF, CHUNK, D), jnp.float32),
        pltpu.VMEM_SHARED((NPAD, D), jnp.float32),
    ] + [pltpu.SemaphoreType.DMA] * (3 * NBUF),
)
def _seg_sum(x_hbm, src_hbm, dst_hbm, out_hbm, idx, rows, aggsh, *sems):
    _seg_sum_body(x_hbm, src_hbm, dst_hbm, out_hbm, idx, rows, aggsh, *sems)


def _dense_body(agg_ref, x_ref, w_ref, b_ref, g_ref, be_ref, sn_ref, o_ref):
    a = agg_ref[...]
    agg = a[0, :N] + a[1, :N]
    h = jnp.dot(agg, w_ref[...], preferred_element_type=jnp.float32)
    h = (h + b_ref[...]) * sn_ref[...]
    mean = jnp.mean(h, axis=0, keepdims=True)
    var = jnp.mean((h - mean) * (h - mean), axis=0, keepdims=True)
    h = (h - mean) * lax.rsqrt(var + EPS) * g_ref[...] + be_ref[...]
    o_ref[...] = x_ref[...] + jnp.maximum(h, 0.0)


def kernel(x, edge_index, snorm_n, W, b, gamma, beta):
    src = edge_index[0].astype(jnp.int32).reshape(NW, NCHUNK, CHUNK)
    dst = edge_index[1].astype(jnp.int32).reshape(NW, NCHUNK, CHUNK)
    agg2 = _seg_sum(x, src, dst)
    return pl.pallas_call(
        _dense_body,
        out_shape=jax.ShapeDtypeStruct((N, D), jnp.float32),
    )(agg2, x, W, b.reshape(1, D), gamma.reshape(1, D),
      beta.reshape(1, D), snorm_n)


# speedup vs baseline: 1.0962x; 1.0962x over previous
import functools

import jax
import jax.numpy as jnp
from jax import lax
from jax.experimental import pallas as pl
from jax.experimental.pallas import tpu as pltpu
from jax.experimental.pallas import tpu_sc as plsc

N = 10000
E = 320000
D = 128
EPS = 1e-5

NC = 2
NS = 16
NW = NC * NS
CHUNK = 80
EPW = E // NW
NCHUNK = EPW // CHUNK
NPAD = 10240
RPS = NPAD // NS


def _seg_sum_body(x_hbm, src_hbm, dst_hbm, out_hbm,
                  sidx, didx, rows, aggsh, sem_a, sem_b, sem_da, sem_db):
    c = lax.axis_index("c")
    s = lax.axis_index("s")
    wid = c * NS + s

    zero = jnp.zeros((16,), jnp.float32)

    ra = rows.at[0]
    rb = rows.at[1]

    def zrow(i, carry):
        for j in range(D // 16):
            rows[0, i, pl.ds(j * 16, 16)] = zero
        return carry

    lax.fori_loop(0, CHUNK, zrow, 0)
    zbase = s * RPS
    for t in range(RPS // CHUNK):
        pltpu.sync_copy(ra, aggsh.at[pl.ds(zbase + t * CHUNK, CHUNK)])

    pltpu.sync_copy(src_hbm.at[wid], sidx)
    plsc.subcore_barrier()

    def gstart(j, buf, sem):
        pltpu.async_copy(x_hbm.at[sidx.at[j]], buf, sem)

    def gwait(j, buf, sem):
        pltpu.make_async_copy(x_hbm.at[sidx.at[j]], buf, sem).wait()

    def dstart(j, b, sem):
        pltpu.async_copy(dst_hbm.at[wid, j], didx.at[b], sem)

    def dwait(j, b, sem):
        pltpu.make_async_copy(dst_hbm.at[wid, j], didx.at[b], sem).wait()

    def scat(buf, b):
        pltpu.sync_copy(buf, aggsh.at[didx.at[b]], add=True)

    pltpu.sync_copy(dst_hbm.at[wid, 0], didx.at[0])
    pltpu.sync_copy(dst_hbm.at[wid, 1], didx.at[1])
    gstart(0, ra, sem_a)
    gstart(1, rb, sem_b)

    def body(k, carry):
        j = 2 * k
        gwait(j, ra, sem_a)
        scat(ra, 0)
        dstart(j + 2, 0, sem_da)
        gstart(j + 2, ra, sem_a)
        gwait(j + 1, rb, sem_b)
        scat(rb, 1)
        jn = jnp.minimum(j + 3, NCHUNK - 1)
        dstart(jn, 1, sem_db)
        gstart(jn, rb, sem_b)
        dwait(j + 2, 0, sem_da)
        dwait(jn, 1, sem_db)
        return carry

    lax.fori_loop(0, (NCHUNK - 1) // 2, body, 0)
    gwait(NCHUNK - 1, ra, sem_a)
    scat(ra, 0)
    gwait(NCHUNK - 1, rb, sem_b)
    plsc.subcore_barrier()

    pltpu.sync_copy(aggsh.at[pl.ds(s * RPS, RPS)],
                    out_hbm.at[c, pl.ds(s * RPS, RPS)])


@functools.partial(
    pl.kernel,
    out_type=jax.ShapeDtypeStruct((NC, NPAD, D), jnp.float32),
    mesh=plsc.VectorSubcoreMesh(core_axis_name="c", subcore_axis_name="s"),
    scratch_types=[
        pltpu.VMEM((NCHUNK, CHUNK), jnp.int32),
        pltpu.VMEM((2, CHUNK), jnp.int32),
        pltpu.VMEM((2, CHUNK, D), jnp.float32),
        pltpu.VMEM_SHARED((NPAD, D), jnp.float32),
        pltpu.SemaphoreType.DMA,
        pltpu.SemaphoreType.DMA,
        pltpu.SemaphoreType.DMA,
        pltpu.SemaphoreType.DMA,
    ],
)
def _seg_sum(x_hbm, src_hbm, dst_hbm, out_hbm,
             sidx, didx, rows, aggsh, sem_a, sem_b, sem_da, sem_db):
    _seg_sum_body(x_hbm, src_hbm, dst_hbm, out_hbm,
                  sidx, didx, rows, aggsh, sem_a, sem_b, sem_da, sem_db)


def _dense_body(agg_ref, x_ref, w_ref, b_ref, g_ref, be_ref, sn_ref, o_ref):
    a = agg_ref[...]
    agg = a[0, :N] + a[1, :N]
    h = jnp.dot(agg, w_ref[...], preferred_element_type=jnp.float32)
    h = (h + b_ref[...]) * sn_ref[...]
    mean = jnp.mean(h, axis=0, keepdims=True)
    var = jnp.mean((h - mean) * (h - mean), axis=0, keepdims=True)
    h = (h - mean) * lax.rsqrt(var + EPS) * g_ref[...] + be_ref[...]
    o_ref[...] = x_ref[...] + jnp.maximum(h, 0.0)


def kernel(x, edge_index, snorm_n, W, b, gamma, beta):
    src = edge_index[0].astype(jnp.int32).reshape(NW, NCHUNK, CHUNK)
    dst = edge_index[1].astype(jnp.int32).reshape(NW, NCHUNK, CHUNK)
    agg2 = _seg_sum(x, src, dst)
    return pl.pallas_call(
        _dense_body,
        out_shape=jax.ShapeDtypeStruct((N, D), jnp.float32),
    )(agg2, x, W, b.reshape(1, D), gamma.reshape(1, D),
      beta.reshape(1, D), snorm_n)


# speedup vs baseline: 1.2015x; 1.0961x over previous
import functools

import jax
import jax.numpy as jnp
from jax import lax
from jax.experimental import pallas as pl
from jax.experimental.pallas import tpu as pltpu
from jax.experimental.pallas import tpu_sc as plsc

N = 10000
E = 320000
D = 128
EPS = 1e-5

NC = 2
NS = 16
NW = NC * NS
CHUNK = 80
EPW = E // NW
NCHUNK = EPW // CHUNK
NPAD = 10240
RPS = NPAD // NS


def _seg_sum_body(x_hbm, ei_hbm, out_hbm,
                  sidx, didx, rows, aggsh, sem_a, sem_b, sem_da, sem_db):
    c = lax.axis_index("c")
    s = lax.axis_index("s")
    wid = c * NS + s

    zero = jnp.zeros((16,), jnp.float32)

    ra = rows.at[0]
    rb = rows.at[1]

    def zrow(i, carry):
        for j in range(D // 16):
            rows[0, i, pl.ds(j * 16, 16)] = zero
        return carry

    lax.fori_loop(0, CHUNK, zrow, 0)
    zbase = s * RPS
    for t in range(RPS // CHUNK):
        pltpu.sync_copy(ra, aggsh.at[pl.ds(zbase + t * CHUNK, CHUNK)])

    ebase = wid * EPW
    pltpu.sync_copy(ei_hbm.at[pl.ds(ebase, EPW)], sidx)
    plsc.subcore_barrier()

    def gstart(j, buf, sem):
        pltpu.async_copy(x_hbm.at[sidx.at[pl.ds(j * CHUNK, CHUNK)]], buf, sem)

    def gwait(j, buf, sem):
        pltpu.make_async_copy(x_hbm.at[sidx.at[pl.ds(j * CHUNK, CHUNK)]],
                              buf, sem).wait()

    def dstart(j, b, sem):
        pltpu.async_copy(ei_hbm.at[pl.ds(E + ebase + j * CHUNK, CHUNK)],
                         didx.at[b], sem)

    def dwait(j, b, sem):
        pltpu.make_async_copy(ei_hbm.at[pl.ds(E + ebase + j * CHUNK, CHUNK)],
                              didx.at[b], sem).wait()

    def scat(buf, b):
        pltpu.sync_copy(buf, aggsh.at[didx.at[b]], add=True)

    pltpu.sync_copy(ei_hbm.at[pl.ds(E + ebase, CHUNK)], didx.at[0])
    pltpu.sync_copy(ei_hbm.at[pl.ds(E + ebase + CHUNK, CHUNK)], didx.at[1])
    gstart(0, ra, sem_a)
    gstart(1, rb, sem_b)

    def body(k, carry):
        j = 2 * k
        gwait(j, ra, sem_a)
        scat(ra, 0)
        dstart(j + 2, 0, sem_da)
        gstart(j + 2, ra, sem_a)
        gwait(j + 1, rb, sem_b)
        scat(rb, 1)
        jn = jnp.minimum(j + 3, NCHUNK - 1)
        dstart(jn, 1, sem_db)
        gstart(jn, rb, sem_b)
        dwait(j + 2, 0, sem_da)
        dwait(jn, 1, sem_db)
        return carry

    lax.fori_loop(0, (NCHUNK - 1) // 2, body, 0)
    gwait(NCHUNK - 1, ra, sem_a)
    scat(ra, 0)
    gwait(NCHUNK - 1, rb, sem_b)
    plsc.subcore_barrier()

    pltpu.sync_copy(aggsh.at[pl.ds(s * RPS, RPS)],
                    out_hbm.at[c, pl.ds(s * RPS, RPS)])


@functools.partial(
    pl.kernel,
    out_type=jax.ShapeDtypeStruct((NC, NPAD, D), jnp.float32),
    mesh=plsc.VectorSubcoreMesh(core_axis_name="c", subcore_axis_name="s"),
    scratch_types=[
        pltpu.VMEM((EPW,), jnp.int32),
        pltpu.VMEM((2, CHUNK), jnp.int32),
        pltpu.VMEM((2, CHUNK, D), jnp.float32),
        pltpu.VMEM_SHARED((NPAD, D), jnp.float32),
        pltpu.SemaphoreType.DMA,
        pltpu.SemaphoreType.DMA,
        pltpu.SemaphoreType.DMA,
        pltpu.SemaphoreType.DMA,
    ],
)
def _seg_sum(x_hbm, ei_hbm, out_hbm,
             sidx, didx, rows, aggsh, sem_a, sem_b, sem_da, sem_db):
    _seg_sum_body(x_hbm, ei_hbm, out_hbm,
                  sidx, didx, rows, aggsh, sem_a, sem_b, sem_da, sem_db)


def _dense_body(agg_ref, x_ref, w_ref, b_ref, g_ref, be_ref, sn_ref, o_ref):
    a = agg_ref[...]
    agg = a[0, :N] + a[1, :N]
    h = jnp.dot(agg, w_ref[...], preferred_element_type=jnp.float32)
    h = (h + b_ref[...]) * sn_ref[...]
    mean = jnp.mean(h, axis=0, keepdims=True)
    var = jnp.mean((h - mean) * (h - mean), axis=0, keepdims=True)
    h = (h - mean) * lax.rsqrt(var + EPS) * g_ref[...] + be_ref[...]
    o_ref[...] = x_ref[...] + jnp.maximum(h, 0.0)


def kernel(x, edge_index, snorm_n, W, b, gamma, beta):
    agg2 = _seg_sum(x, edge_index.astype(jnp.int32).reshape(2 * E))
    return pl.pallas_call(
        _dense_body,
        out_shape=jax.ShapeDtypeStruct((N, D), jnp.float32),
    )(agg2, x, W, b.reshape(1, D), gamma.reshape(1, D),
      beta.reshape(1, D), snorm_n)
